# baseline (speedup 1.0000x reference)
import functools

import jax
import jax.numpy as jnp
from jax import lax
from jax.experimental import pallas as pl
from jax.experimental.pallas import tpu as pltpu
from jax.experimental.pallas import tpu_sc as plsc

B = 16384
D = 32
L = 16
NC = 2
NS = 16
NW = NC * NS
BPW = B // NW
GROUPS = BPW // L

_mesh = plsc.VectorSubcoreMesh(core_axis_name="c", subcore_axis_name="s")


@functools.partial(
    pl.kernel,
    mesh=_mesh,
    out_type=jax.ShapeDtypeStruct((B,), jnp.float32),
    compiler_params=pltpu.CompilerParams(
        needs_layout_passes=False, use_tc_tiling_on_sc=False
    ),
    scratch_types=[
        pltpu.VMEM((BPW,), jnp.int32),
        pltpu.VMEM((BPW,), jnp.int32),
        pltpu.VMEM((BPW, D), jnp.float32),
        pltpu.VMEM((BPW, D), jnp.float32),
        pltpu.VMEM((BPW,), jnp.float32),
        pltpu.VMEM((BPW,), jnp.float32),
        pltpu.VMEM((BPW,), jnp.float32),
        pltpu.SemaphoreType.DMA,
        pltpu.SemaphoreType.DMA,
        pltpu.SemaphoreType.DMA,
        pltpu.SemaphoreType.DMA,
    ],
)
def _mf_sc(users_hbm, items_hbm, ut_hbm, it_hbm, ub_hbm, ib_hbm, out_hbm,
           uidx, iidx, urows, irows, ubv, ibv, res,
           sem_u, sem_i, sem_ub, sem_ib):
    wid = lax.axis_index("s") * NC + lax.axis_index("c")
    base = wid * BPW

    pltpu.sync_copy(users_hbm.at[pl.ds(base, BPW)], uidx)
    pltpu.sync_copy(items_hbm.at[pl.ds(base, BPW)], iidx)

    cu = pltpu.async_copy(ut_hbm.at[uidx], urows, sem_u)
    ci = pltpu.async_copy(it_hbm.at[iidx], irows, sem_i)
    cub = pltpu.async_copy(ub_hbm.at[uidx], ubv, sem_ub)
    cib = pltpu.async_copy(ib_hbm.at[iidx], ibv, sem_ib)
    cu.wait()
    ci.wait()
    cub.wait()
    cib.wait()

    def group_body(g, carry):
        row = lax.iota(jnp.int32, L) + g * L
        acc = ubv[pl.ds(g * L, L)] + ibv[pl.ds(g * L, L)]
        for j in range(D):
            col = jnp.full((L,), j, jnp.int32)
            u = plsc.load_gather(urows, [row, col])
            v = plsc.load_gather(irows, [row, col])
            acc = acc + u * v
        res[pl.ds(g * L, L)] = acc
        return carry

    lax.fori_loop(0, GROUPS, group_body, 0)

    pltpu.sync_copy(res, out_hbm.at[pl.ds(base, BPW)])


def kernel(users, items, user_table, item_table, user_bias, item_bias):
    return _mf_sc(users, items, user_table, item_table, user_bias, item_bias)
